# baseline (speedup 1.0000x reference)
import jax
import jax.numpy as jnp
from jax import lax
from jax.experimental import pallas as pl
from jax.experimental.pallas import tpu as pltpu
from jax.experimental.pallas import tpu_sc as plsc

N_ENT = 10000
D = 128
NSPLIT = 4
W = D // NSPLIT
E = 320000
NREL = 474
NRELP = 480

NC, NS, L = 2, 16, 16
NT = NC * NS
NP = 10240
CH = 128
NCHUNK = 80
EPT = NCHUNK * CH
EP = NT * EPT
RPT = NP // NS

_mesh = plsc.VectorSubcoreMesh(
    core_axis_name="c", subcore_axis_name="s", num_cores=NC, num_subcores=NS)
_sc_params = pltpu.CompilerParams(use_tc_tiling_on_sc=False)


def _count_body(row_hbm, col_hbm, deg_out, cnt_out,
                ri, ci, ones_v, idx_v, stage_v, hist_acc):
    c = lax.axis_index("c")
    s = lax.axis_index("s")
    ebase = (c * NS + s) * EPT

    pltpu.sync_copy(row_hbm.at[pl.ds(ebase, EPT)], ri)
    pltpu.sync_copy(col_hbm.at[pl.ds(ebase, EPT)], ci)

    def fill(i, _):
        ones_v[i, :] = jnp.ones((L,), jnp.float32)
        stage_v[i, :] = jnp.zeros((L,), jnp.float32)
        return 0
    lax.fori_loop(0, CH, fill, 0)

    for src, out in ((ri, deg_out), (ci, cnt_out)):
        for k in range(RPT // CH):
            pltpu.sync_copy(stage_v, hist_acc.at[pl.ds(s * RPT + k * CH, CH)])
        plsc.subcore_barrier()

        def chunk(ch, _):
            for g in range(CH // L):
                idx_v[pl.ds(g * L, L)] = src[pl.ds(ch * CH + g * L, L)]
            pltpu.sync_copy(ones_v, hist_acc.at[idx_v], add=True)
            return 0
        lax.fori_loop(0, NCHUNK, chunk, 0)
        plsc.subcore_barrier()

        for k in range(RPT // CH):
            r0 = s * RPT + k * CH
            pltpu.sync_copy(hist_acc.at[pl.ds(r0, CH)], stage_v)
            pltpu.sync_copy(stage_v, out.at[c, pl.ds(r0, CH)])
        plsc.subcore_barrier()


_count_kernel = pl.kernel(
    _count_body,
    out_type=(jax.ShapeDtypeStruct((NC, NP, L), jnp.float32),
              jax.ShapeDtypeStruct((NC, NP, L), jnp.float32)),
    mesh=_mesh,
    compiler_params=_sc_params,
    scratch_types=[
        pltpu.VMEM((EPT,), jnp.int32),
        pltpu.VMEM((EPT,), jnp.int32),
        pltpu.VMEM((CH, L), jnp.float32),
        pltpu.VMEM((CH,), jnp.int32),
        pltpu.VMEM((CH, L), jnp.float32),
        pltpu.VMEM_SHARED((NP, L), jnp.float32),
    ],
)


def _edge_body(xp4_hbm, ee4_hbm, rel4_hbm,
               row_hbm, col_hbm, typ_hbm, ids_hbm,
               part0, part1, part2, part3,
               ri, ci, ti, ii, rel_v,
               xg0, xg1, xg2, xg3, eg0, eg1, eg2, eg3,
               rc0, rc1, rc2, rc3, ic0, ic1, ic2, ic3,
               cc0, cc1, cc2, cc3, acc,
               sx0, sx1, sx2, sx3, se0, se1, se2, se3,
               ss0, ss1, ss2, ss3):
    parts = (part0, part1, part2, part3)
    c = lax.axis_index("c")
    s = lax.axis_index("s")
    ebase = (c * NS + s) * EPT
    NB = 4
    xg = (xg0, xg1, xg2, xg3)
    eg = (eg0, eg1, eg2, eg3)
    rc = (rc0, rc1, rc2, rc3)
    ic = (ic0, ic1, ic2, ic3)
    cc = (cc0, cc1, cc2, cc3)
    sx = (sx0, sx1, sx2, sx3)
    se = (se0, se1, se2, se3)
    ss = (ss0, ss1, ss2, ss3)

    pltpu.sync_copy(row_hbm.at[pl.ds(ebase, EPT)], ri)
    pltpu.sync_copy(col_hbm.at[pl.ds(ebase, EPT)], ci)
    pltpu.sync_copy(typ_hbm.at[pl.ds(ebase, EPT)], ti)
    pltpu.sync_copy(ids_hbm.at[pl.ds(ebase, EPT)], ii)

    def zero_own_rows():
        def fz(i, _):
            for l in range(W // L):
                xg0[i, pl.ds(l * L, L)] = jnp.zeros((L,), jnp.float32)
            return 0
        lax.fori_loop(0, CH, fz, 0)
        for k in range(RPT // CH):
            pltpu.sync_copy(xg0, acc.at[pl.ds(s * RPT + k * CH, CH)])

    def prep(n, b, h):
        for g in range(CH // L):
            dst = pl.ds(g * L, L)
            src = pl.ds(n * CH + g * L, L)
            rc[b][dst] = ri[src] * NSPLIT + h
            ic[b][dst] = ii[src] * NSPLIT + h
            cc[b][dst] = ci[src]

    def fire(b):
        pltpu.async_copy(xp4_hbm.at[rc[b]], xg[b], sx[b])
        pltpu.async_copy(ee4_hbm.at[ic[b]], eg[b], se[b])

    def wait(b):
        pltpu.make_async_copy(xp4_hbm.at[rc[b]], xg[b], sx[b]).wait()
        pltpu.make_async_copy(ee4_hbm.at[ic[b]], eg[b], se[b]).wait()

    zero_own_rows()
    plsc.subcore_barrier()

    def half(h, _):
        pltpu.sync_copy(rel4_hbm.at[pl.ds(h * NRELP, NRELP)], rel_v)
        for k in range(NB - 1):
            prep(k, k, h)
            fire(k)

        def chunk(ch, b):
            pf = (b + NB - 1) % NB
            wait(b)

            @pl.when(ch + NB - 1 < NCHUNK)
            def _():
                prep(ch + NB - 1, pf, h)
                @pl.when(ch >= 1)
                def _():
                    pltpu.make_async_copy(xg[pf], acc.at[cc[pf]],
                                          ss[pf]).wait()
                fire(pf)

            def group(g, _):
                tvec = ti[pl.ds(ch * CH + g * L, L)]
                for j in range(L):
                    tt = tvec[j]
                    e = g * L + j
                    for l in range(W // L):
                        sl = pl.ds(l * L, L)
                        xg[b][e, sl] = (xg[b][e, sl] * eg[b][e, sl]
                                        * rel_v[tt, sl])
                return 0
            lax.fori_loop(0, CH // L, group, 0)
            pltpu.async_copy(xg[b], acc.at[cc[b]], ss[b], add=True)
            return 0

        def quad(p, _):
            for b in range(NB):
                chunk(NB * p + b, b)
            return 0
        lax.fori_loop(0, NCHUNK // NB, quad, 0)
        for b in range(NB):
            pltpu.make_async_copy(xg[b], acc.at[cc[b]], ss[b]).wait()
        plsc.subcore_barrier()

        for k in range(RPT // CH):
            r0 = s * RPT + k * CH
            pltpu.sync_copy(acc.at[pl.ds(r0, CH)], eg0)

            for q in range(NSPLIT):
                @pl.when(h == q)
                def _(q=q):
                    pltpu.sync_copy(eg0, parts[q].at[c, pl.ds(r0, CH)])
        zero_own_rows()
        plsc.subcore_barrier()
        return 0

    lax.fori_loop(0, NSPLIT, half, 0)


_edge_kernel = pl.kernel(
    _edge_body,
    out_type=tuple(jax.ShapeDtypeStruct((NC, NP, W), jnp.float32)
                   for _ in range(NSPLIT)),
    mesh=_mesh,
    compiler_params=_sc_params,
    scratch_types=[
        pltpu.VMEM((EPT,), jnp.int32),
        pltpu.VMEM((EPT,), jnp.int32),
        pltpu.VMEM((EPT,), jnp.int32),
        pltpu.VMEM((EPT,), jnp.int32),
        pltpu.VMEM((NRELP, W), jnp.float32),
    ] + [pltpu.VMEM((CH, W), jnp.float32) for _ in range(8)]
    + [pltpu.VMEM((CH,), jnp.int32) for _ in range(12)]
    + [pltpu.VMEM_SHARED((NP, W), jnp.float32)]
    + [pltpu.SemaphoreType.DMA for _ in range(12)],
)


def _prep_body(deg_ref, cnt_ref, x_ref, xp_ref, t_ref, u_ref, sc_ref):
    deg = deg_ref[0, :, 0:1] + deg_ref[1, :, 0:1] + 1.0
    cnt = cnt_ref[0, :, 0:1] + cnt_ref[1, :, 0:1] + 1.0
    t = lax.rsqrt(deg)
    invc = 1.0 / cnt
    t_ref[...] = t
    u_ref[...] = t * invc
    sc_ref[...] = invc / deg
    xp_ref[...] = t * x_ref[...]


_GRID = 8
_BR = NP // _GRID

_prep = pl.pallas_call(
    _prep_body,
    grid=(_GRID,),
    in_specs=[
        pl.BlockSpec((NC, _BR, L), lambda i: (0, i, 0)),
        pl.BlockSpec((NC, _BR, L), lambda i: (0, i, 0)),
        pl.BlockSpec((_BR, D), lambda i: (i, 0)),
    ],
    out_specs=(pl.BlockSpec((_BR, D), lambda i: (i, 0)),
               pl.BlockSpec((_BR, 1), lambda i: (i, 0)),
               pl.BlockSpec((_BR, 1), lambda i: (i, 0)),
               pl.BlockSpec((_BR, 1), lambda i: (i, 0))),
    out_shape=(jax.ShapeDtypeStruct((NP, D), jnp.float32),
               jax.ShapeDtypeStruct((NP, 1), jnp.float32),
               jax.ShapeDtypeStruct((NP, 1), jnp.float32),
               jax.ShapeDtypeStruct((NP, 1), jnp.float32)),
)


def _combine_body(p0_ref, p1_ref, p2_ref, p3_ref, x_ref, u_ref, sc_ref,
                  t_ref, step_ref, y_ref, yp_ref):
    u = u_ref[...]
    scv = sc_ref[...]
    prefs = (p0_ref, p1_ref, p2_ref, p3_ref)
    yq = [u * (prefs[q][0] + prefs[q][1]) + scv * x_ref[:, q * W:(q + 1) * W]
          for q in range(NSPLIT)]
    y = jnp.concatenate(yq, axis=1)
    y = jnp.where(step_ref[0, 0] == 1, jnp.maximum(y, 0.0), y)
    y_ref[...] = y
    yp_ref[...] = t_ref[...] * y


_combine = pl.pallas_call(
    _combine_body,
    grid=(_GRID,),
    in_specs=[pl.BlockSpec((NC, _BR, W), lambda i: (0, i, 0))
              for _ in range(NSPLIT)] + [
        pl.BlockSpec((_BR, D), lambda i: (i, 0)),
        pl.BlockSpec((_BR, 1), lambda i: (i, 0)),
        pl.BlockSpec((_BR, 1), lambda i: (i, 0)),
        pl.BlockSpec((_BR, 1), lambda i: (i, 0)),
        pl.BlockSpec((1, 1), lambda i: (0, 0)),
    ],
    out_specs=(pl.BlockSpec((_BR, D), lambda i: (i, 0)),
               pl.BlockSpec((_BR, D), lambda i: (i, 0))),
    out_shape=(jax.ShapeDtypeStruct((NP, D), jnp.float32),
               jax.ShapeDtypeStruct((NP, D), jnp.float32)),
)


def kernel(edge_index, edge_type, edge_ids,
           entity_embedding, relation_embedding, edge_embedding):
    npad = EP - E
    ar = jnp.arange(npad, dtype=jnp.int32)
    pad_nodes = N_ENT + (ar % (NP - N_ENT))
    rowp = jnp.concatenate([edge_index[0].astype(jnp.int32), pad_nodes])
    colp = jnp.concatenate([edge_index[1].astype(jnp.int32), pad_nodes])
    typp = jnp.concatenate([edge_type.astype(jnp.int32), ar % NREL])
    idsp = jnp.concatenate([edge_ids.astype(jnp.int32), ar % E])

    x0 = jnp.concatenate(
        [entity_embedding, jnp.zeros((NP - N_ENT, D), jnp.float32)], axis=0)
    ee4 = edge_embedding.reshape(NSPLIT * E, W)
    relp = jnp.concatenate(
        [relation_embedding,
         jnp.zeros((NRELP - NREL, D), jnp.float32)], axis=0)
    rel4 = relp.reshape(NRELP, NSPLIT, W).transpose(1, 0, 2).reshape(
        NSPLIT * NRELP, W)

    deg16, cnt16 = _count_kernel(rowp, colp)
    xp0, t, u, sc = _prep(deg16, cnt16, x0)

    x, xp = x0, xp0
    for step in range(3):
        ps = _edge_kernel(xp.reshape(NSPLIT * NP, W), ee4, rel4,
                          rowp, colp, typp, idsp)
        x, xp = _combine(*ps, x, u, sc, t,
                         jnp.full((1, 1), step, jnp.int32))
    return x[:N_ENT]


# speedup vs baseline: 1.0344x; 1.0344x over previous
import jax
import jax.numpy as jnp
from jax import lax
from jax.experimental import pallas as pl
from jax.experimental.pallas import tpu as pltpu
from jax.experimental.pallas import tpu_sc as plsc

N_ENT = 10000
D = 128
NSPLIT = 4
W = D // NSPLIT
E = 320000
NREL = 474
NRELP = 480

NC, NS, L = 2, 16, 16
NT = NC * NS
NP = 10240
CH = 128
NCHUNK = 80
EPT = NCHUNK * CH
EP = NT * EPT
RPT = NP // NS

_mesh = plsc.VectorSubcoreMesh(
    core_axis_name="c", subcore_axis_name="s", num_cores=NC, num_subcores=NS)
_sc_params = pltpu.CompilerParams(use_tc_tiling_on_sc=False)


def _count_body(row_hbm, col_hbm, deg_out, cnt_out,
                ri, ci, ones_v, idx_v, stage_v, hist_acc):
    c = lax.axis_index("c")
    s = lax.axis_index("s")
    ebase = (c * NS + s) * EPT

    pltpu.sync_copy(row_hbm.at[pl.ds(ebase, EPT)], ri)
    pltpu.sync_copy(col_hbm.at[pl.ds(ebase, EPT)], ci)

    def fill(i, _):
        ones_v[i, :] = jnp.ones((L,), jnp.float32)
        stage_v[i, :] = jnp.zeros((L,), jnp.float32)
        return 0
    lax.fori_loop(0, CH, fill, 0)

    for src, out in ((ri, deg_out), (ci, cnt_out)):
        for k in range(RPT // CH):
            pltpu.sync_copy(stage_v, hist_acc.at[pl.ds(s * RPT + k * CH, CH)])
        plsc.subcore_barrier()

        def chunk(ch, _):
            for g in range(CH // L):
                idx_v[pl.ds(g * L, L)] = src[pl.ds(ch * CH + g * L, L)]
            pltpu.sync_copy(ones_v, hist_acc.at[idx_v], add=True)
            return 0
        lax.fori_loop(0, NCHUNK, chunk, 0)
        plsc.subcore_barrier()

        for k in range(RPT // CH):
            r0 = s * RPT + k * CH
            pltpu.sync_copy(hist_acc.at[pl.ds(r0, CH)], stage_v)
            pltpu.sync_copy(stage_v, out.at[c, pl.ds(r0, CH)])
        plsc.subcore_barrier()


_count_kernel = pl.kernel(
    _count_body,
    out_type=(jax.ShapeDtypeStruct((NC, NP, L), jnp.float32),
              jax.ShapeDtypeStruct((NC, NP, L), jnp.float32)),
    mesh=_mesh,
    compiler_params=_sc_params,
    scratch_types=[
        pltpu.VMEM((EPT,), jnp.int32),
        pltpu.VMEM((EPT,), jnp.int32),
        pltpu.VMEM((CH, L), jnp.float32),
        pltpu.VMEM((CH,), jnp.int32),
        pltpu.VMEM((CH, L), jnp.float32),
        pltpu.VMEM_SHARED((NP, L), jnp.float32),
    ],
)


def _edge_body(xp4_hbm, ee4_hbm, rel4_hbm,
               row_hbm, col_hbm, typ_hbm, ids_hbm,
               part0, part1, part2, part3,
               ri, ci, ti, ii, rel_v, xg0, xg1, eg0, eg1,
               rc0, rc1, ic0, ic1, cc0, cc1, acc,
               sx0, sx1, se0, se1, ss0, ss1):
    parts = (part0, part1, part2, part3)
    c = lax.axis_index("c")
    s = lax.axis_index("s")
    ebase = (c * NS + s) * EPT
    xg = (xg0, xg1)
    eg = (eg0, eg1)
    rc = (rc0, rc1)
    ic = (ic0, ic1)
    cc = (cc0, cc1)
    sx = (sx0, sx1)
    se = (se0, se1)
    ss = (ss0, ss1)

    pltpu.sync_copy(row_hbm.at[pl.ds(ebase, EPT)], ri)
    pltpu.sync_copy(col_hbm.at[pl.ds(ebase, EPT)], ci)
    pltpu.sync_copy(typ_hbm.at[pl.ds(ebase, EPT)], ti)
    pltpu.sync_copy(ids_hbm.at[pl.ds(ebase, EPT)], ii)

    def zero_own_rows():
        def fz(i, _):
            for l in range(W // L):
                xg0[i, pl.ds(l * L, L)] = jnp.zeros((L,), jnp.float32)
            return 0
        lax.fori_loop(0, CH, fz, 0)
        for k in range(RPT // CH):
            pltpu.sync_copy(xg0, acc.at[pl.ds(s * RPT + k * CH, CH)])

    def prep(n, b, h):
        for g in range(CH // L):
            dst = pl.ds(g * L, L)
            src = pl.ds(n * CH + g * L, L)
            rc[b][dst] = ri[src] * NSPLIT + h
            ic[b][dst] = ii[src] * NSPLIT + h
            cc[b][dst] = ci[src]

    def fire(b):
        return (pltpu.async_copy(xp4_hbm.at[rc[b]], xg[b], sx[b]),
                pltpu.async_copy(ee4_hbm.at[ic[b]], eg[b], se[b]))

    def wait(b):
        pltpu.make_async_copy(xp4_hbm.at[rc[b]], xg[b], sx[b]).wait()
        pltpu.make_async_copy(ee4_hbm.at[ic[b]], eg[b], se[b]).wait()

    zero_own_rows()
    plsc.subcore_barrier()

    def half(h, _):
        pltpu.sync_copy(rel4_hbm.at[pl.ds(h * NRELP, NRELP)], rel_v)
        prep(0, 0, h)
        fire(0)

        def chunk(ch, b):
            nb = 1 - b
            wait(b)

            @pl.when(ch + 1 < NCHUNK)
            def _():
                prep(ch + 1, nb, h)
                @pl.when(ch >= 1)
                def _():
                    pltpu.make_async_copy(xg[nb], acc.at[cc[nb]],
                                          ss[nb]).wait()
                fire(nb)

            def group(g, _):
                tvec = ti[pl.ds(ch * CH + g * L, L)]
                for j in range(L):
                    tt = tvec[j]
                    e = g * L + j
                    for l in range(W // L):
                        sl = pl.ds(l * L, L)
                        xg[b][e, sl] = (xg[b][e, sl] * eg[b][e, sl]
                                        * rel_v[tt, sl])
                return 0
            lax.fori_loop(0, CH // L, group, 0)
            pltpu.async_copy(xg[b], acc.at[cc[b]], ss[b], add=True)
            return 0

        def pair(p, _):
            chunk(2 * p, 0)
            chunk(2 * p + 1, 1)
            return 0
        lax.fori_loop(0, NCHUNK // 2, pair, 0)
        pltpu.make_async_copy(xg0, acc.at[cc0], ss0).wait()
        pltpu.make_async_copy(xg1, acc.at[cc1], ss1).wait()
        plsc.subcore_barrier()

        for k in range(RPT // CH):
            r0 = s * RPT + k * CH
            pltpu.sync_copy(acc.at[pl.ds(r0, CH)], eg0)

            for q in range(NSPLIT):
                @pl.when(h == q)
                def _(q=q):
                    pltpu.sync_copy(eg0, parts[q].at[c, pl.ds(r0, CH)])
        zero_own_rows()
        plsc.subcore_barrier()
        return 0

    lax.fori_loop(0, NSPLIT, half, 0)


_edge_kernel = pl.kernel(
    _edge_body,
    out_type=tuple(jax.ShapeDtypeStruct((NC, NP, W), jnp.float32)
                   for _ in range(NSPLIT)),
    mesh=_mesh,
    compiler_params=_sc_params,
    scratch_types=[
        pltpu.VMEM((EPT,), jnp.int32),
        pltpu.VMEM((EPT,), jnp.int32),
        pltpu.VMEM((EPT,), jnp.int32),
        pltpu.VMEM((EPT,), jnp.int32),
        pltpu.VMEM((NRELP, W), jnp.float32),
        pltpu.VMEM((CH, W), jnp.float32),
        pltpu.VMEM((CH, W), jnp.float32),
        pltpu.VMEM((CH, W), jnp.float32),
        pltpu.VMEM((CH, W), jnp.float32),
        pltpu.VMEM((CH,), jnp.int32),
        pltpu.VMEM((CH,), jnp.int32),
        pltpu.VMEM((CH,), jnp.int32),
        pltpu.VMEM((CH,), jnp.int32),
        pltpu.VMEM((CH,), jnp.int32),
        pltpu.VMEM((CH,), jnp.int32),
        pltpu.VMEM_SHARED((NP, W), jnp.float32),
        pltpu.SemaphoreType.DMA,
        pltpu.SemaphoreType.DMA,
        pltpu.SemaphoreType.DMA,
        pltpu.SemaphoreType.DMA,
        pltpu.SemaphoreType.DMA,
        pltpu.SemaphoreType.DMA,
    ],
)


def _prep_body(deg_ref, cnt_ref, x_ref, xp_ref, t_ref, u_ref, sc_ref):
    deg = deg_ref[0, :, 0:1] + deg_ref[1, :, 0:1] + 1.0
    cnt = cnt_ref[0, :, 0:1] + cnt_ref[1, :, 0:1] + 1.0
    t = lax.rsqrt(deg)
    invc = 1.0 / cnt
    t_ref[...] = t
    u_ref[...] = t * invc
    sc_ref[...] = invc / deg
    xp_ref[...] = t * x_ref[...]


_GRID = 8
_BR = NP // _GRID

_prep = pl.pallas_call(
    _prep_body,
    grid=(_GRID,),
    in_specs=[
        pl.BlockSpec((NC, _BR, L), lambda i: (0, i, 0)),
        pl.BlockSpec((NC, _BR, L), lambda i: (0, i, 0)),
        pl.BlockSpec((_BR, D), lambda i: (i, 0)),
    ],
    out_specs=(pl.BlockSpec((_BR, D), lambda i: (i, 0)),
               pl.BlockSpec((_BR, 1), lambda i: (i, 0)),
               pl.BlockSpec((_BR, 1), lambda i: (i, 0)),
               pl.BlockSpec((_BR, 1), lambda i: (i, 0))),
    out_shape=(jax.ShapeDtypeStruct((NP, D), jnp.float32),
               jax.ShapeDtypeStruct((NP, 1), jnp.float32),
               jax.ShapeDtypeStruct((NP, 1), jnp.float32),
               jax.ShapeDtypeStruct((NP, 1), jnp.float32)),
)


def _combine_body(p0_ref, p1_ref, p2_ref, p3_ref, x_ref, u_ref, sc_ref,
                  t_ref, step_ref, y_ref, yp_ref):
    u = u_ref[...]
    scv = sc_ref[...]
    prefs = (p0_ref, p1_ref, p2_ref, p3_ref)
    yq = [u * (prefs[q][0] + prefs[q][1]) + scv * x_ref[:, q * W:(q + 1) * W]
          for q in range(NSPLIT)]
    y = jnp.concatenate(yq, axis=1)
    y = jnp.where(step_ref[0, 0] == 1, jnp.maximum(y, 0.0), y)
    y_ref[...] = y
    yp_ref[...] = t_ref[...] * y


_combine = pl.pallas_call(
    _combine_body,
    grid=(_GRID,),
    in_specs=[pl.BlockSpec((NC, _BR, W), lambda i: (0, i, 0))
              for _ in range(NSPLIT)] + [
        pl.BlockSpec((_BR, D), lambda i: (i, 0)),
        pl.BlockSpec((_BR, 1), lambda i: (i, 0)),
        pl.BlockSpec((_BR, 1), lambda i: (i, 0)),
        pl.BlockSpec((_BR, 1), lambda i: (i, 0)),
        pl.BlockSpec((1, 1), lambda i: (0, 0)),
    ],
    out_specs=(pl.BlockSpec((_BR, D), lambda i: (i, 0)),
               pl.BlockSpec((_BR, D), lambda i: (i, 0))),
    out_shape=(jax.ShapeDtypeStruct((NP, D), jnp.float32),
               jax.ShapeDtypeStruct((NP, D), jnp.float32)),
)


def kernel(edge_index, edge_type, edge_ids,
           entity_embedding, relation_embedding, edge_embedding):
    npad = EP - E
    ar = jnp.arange(npad, dtype=jnp.int32)
    pad_nodes = N_ENT + (ar % (NP - N_ENT))
    rowp = jnp.concatenate([edge_index[0].astype(jnp.int32), pad_nodes])
    colp = jnp.concatenate([edge_index[1].astype(jnp.int32), pad_nodes])
    typp = jnp.concatenate([edge_type.astype(jnp.int32), ar % NREL])
    idsp = jnp.concatenate([edge_ids.astype(jnp.int32), ar % E])

    x0 = jnp.concatenate(
        [entity_embedding, jnp.zeros((NP - N_ENT, D), jnp.float32)], axis=0)
    ee4 = edge_embedding.reshape(NSPLIT * E, W)
    relp = jnp.concatenate(
        [relation_embedding,
         jnp.zeros((NRELP - NREL, D), jnp.float32)], axis=0)
    rel4 = relp.reshape(NRELP, NSPLIT, W).transpose(1, 0, 2).reshape(
        NSPLIT * NRELP, W)

    deg16, cnt16 = _count_kernel(rowp, colp)
    xp0, t, u, sc = _prep(deg16, cnt16, x0)

    x, xp = x0, xp0
    for step in range(3):
        ps = _edge_kernel(xp.reshape(NSPLIT * NP, W), ee4, rel4,
                          rowp, colp, typp, idsp)
        x, xp = _combine(*ps, x, u, sc, t,
                         jnp.full((1, 1), step, jnp.int32))
    return x[:N_ENT]
